# Initial kernel scaffold; baseline (speedup 1.0000x reference)
import jax
import jax.numpy as jnp
from jax.experimental import pallas as pl


def kernel(x, w1, g1, b1, m1, v1, w2, g2, b2, m2, v2, wds, bds, gds, bds_bn, mds, vds):
    raise NotImplementedError("write your pallas kernel here")


# speedup vs baseline: 1.8649x; 1.8649x over previous
import functools

import jax
import jax.numpy as jnp
from jax import lax
from jax.experimental import pallas as pl
from jax.experimental.pallas import tpu as pltpu

_EPS = 1e-5


def _fold(gamma, beta, mean, var):
    s = gamma / jnp.sqrt(var + _EPS)
    return s, beta - mean * s


def _block_kernel(x_ref, w1_ref, b1_ref, w2_ref, b2_ref, o_ref,
                  xp_ref, y1p_ref, *, H, W, C):
    HW = H * W
    lead = W + 1
    data0 = lead
    data1 = lead + HW

    col = lax.broadcasted_iota(jnp.int32, (1, HW), 1) % W
    mask_l = col != 0
    mask_r = col != W - 1

    def cols_from(slab_ref):
        taps = []
        for kh in range(3):
            for kw in range(3):
                t = slab_ref[:, pl.ds(kh * W + kw, HW)]
                if kw == 0:
                    t = jnp.where(mask_l, t, jnp.bfloat16(0))
                elif kw == 2:
                    t = jnp.where(mask_r, t, jnp.bfloat16(0))
                taps.append(t)
        return jnp.concatenate(taps, axis=0)

    x = x_ref[0]
    xp_ref[:, pl.ds(0, data0)] = jnp.zeros((C, data0), jnp.bfloat16)
    xp_ref[:, pl.ds(data0, HW)] = x.astype(jnp.bfloat16)
    xp_ref[:, pl.ds(data1, lead)] = jnp.zeros((C, lead), jnp.bfloat16)

    y1 = jnp.dot(w1_ref[...], cols_from(xp_ref),
                 preferred_element_type=jnp.float32)
    y1 = jnp.maximum(y1 + b1_ref[...], 0.0).astype(jnp.bfloat16)

    y1p_ref[:, pl.ds(0, data0)] = jnp.zeros((C, data0), jnp.bfloat16)
    y1p_ref[:, pl.ds(data0, HW)] = y1
    y1p_ref[:, pl.ds(data1, lead)] = jnp.zeros((C, lead), jnp.bfloat16)

    y2 = jnp.dot(w2_ref[...], cols_from(y1p_ref),
                 preferred_element_type=jnp.float32)
    o_ref[0] = y2 + b2_ref[...] + x


@jax.jit
def _basic_block(x, w1, g1, b1, m1, v1, w2, g2, b2, m2, v2):
    N, C, H, W = x.shape
    HW = H * W
    slab = HW + 2 * (W + 1)

    s1, bb1 = _fold(g1, b1, m1, v1)
    s2, bb2 = _fold(g2, b2, m2, v2)
    w1c = (w1 * s1).reshape(9, C, C).transpose(2, 0, 1)
    w1c = w1c.reshape(C, 9 * C).astype(jnp.bfloat16)
    w2c = (w2 * s2).reshape(9, C, C).transpose(2, 0, 1)
    w2c = w2c.reshape(C, 9 * C).astype(jnp.bfloat16)
    bb1 = bb1.reshape(C, 1).astype(jnp.float32)
    bb2 = bb2.reshape(C, 1).astype(jnp.float32)

    kern = functools.partial(_block_kernel, H=H, W=W, C=C)
    out = pl.pallas_call(
        kern,
        out_shape=jax.ShapeDtypeStruct((N, C, HW), jnp.float32),
        grid=(N,),
        in_specs=[
            pl.BlockSpec((1, C, HW), lambda n: (n, 0, 0)),
            pl.BlockSpec((C, 9 * C), lambda n: (0, 0)),
            pl.BlockSpec((C, 1), lambda n: (0, 0)),
            pl.BlockSpec((C, 9 * C), lambda n: (0, 0)),
            pl.BlockSpec((C, 1), lambda n: (0, 0)),
        ],
        out_specs=pl.BlockSpec((1, C, HW), lambda n: (n, 0, 0)),
        scratch_shapes=[
            pltpu.VMEM((C, slab), jnp.bfloat16),
            pltpu.VMEM((C, slab), jnp.bfloat16),
        ],
        compiler_params=pltpu.CompilerParams(
            dimension_semantics=("parallel",)),
    )(x.reshape(N, C, HW), w1c, bb1, w2c, bb2)
    return out.reshape(N, C, H, W)


def kernel(x, w1, g1, b1, m1, v1, w2, g2, b2, m2, v2,
           wds, bds, gds, bds_bn, mds, vds):
    del wds, bds, gds, bds_bn, mds, vds
    return _basic_block(x, w1, g1, b1, m1, v1, w2, g2, b2, m2, v2)
